# Initial kernel scaffold; baseline (speedup 1.0000x reference)
import jax
import jax.numpy as jnp
from jax.experimental import pallas as pl


def kernel(x_joint, x_grain, ei_jj, ei_jg, ei_gj, ea_jj, ea_jg, ea_gj, params):
    raise NotImplementedError("write your pallas kernel here")


# speedup vs baseline: 19.1159x; 19.1159x over previous
import functools
import jax
import jax.numpy as jnp
from jax import lax
from jax.experimental import pallas as pl
from jax.experimental.pallas import tpu as pltpu
from jax.experimental.pallas import tpu_sc as plsc

H = 32
NJ, NG = 50000, 10000
GATES = ['i', 'f', 'c', 'o']
RPT_J, RPT_G = 3140, 640
ZR = 20
ACC_J, ACC_G = 16 * RPT_J, 16 * RPT_G
EP = {'jj': 98 * 16384, 'jg': 20 * 16384, 'gj': 20 * 16384}
NCH = {'jj': 98, 'jg': 20, 'gj': 20}
NDST = {'jj': NJ, 'jg': NG, 'gj': NJ}


def _sc_pass(d, weighted0, weighted1, count_col):
    mesh = plsc.VectorSubcoreMesh(core_axis_name="c", subcore_axis_name="s")
    f32 = jnp.float32
    out_type = [
        jax.ShapeDtypeStruct((ACC_J, d), f32),
        jax.ShapeDtypeStruct((ACC_G, d), f32),
        jax.ShapeDtypeStruct((ACC_J, d), f32),
        jax.ShapeDtypeStruct((ACC_J, d), f32),
        jax.ShapeDtypeStruct((ACC_G, d), f32),
        jax.ShapeDtypeStruct((ACC_J, d), f32),
    ]
    scratch = [
        pltpu.VMEM_SHARED((ACC_J, d), f32),
        pltpu.VMEM_SHARED((ACC_G, d), f32),
        pltpu.VMEM((8, 128), jnp.int32),
        pltpu.VMEM((8, 128), jnp.int32),
        pltpu.VMEM((8, 128), f32),
        pltpu.VMEM((128, d), f32),
        pltpu.VMEM((ZR, d), f32),
        pltpu.SemaphoreType.DMA,
    ]

    def body(fj0, fg0, fj1, fg1, w0jj, w0jg, w0gj, w1jj, w1jg, w1gj,
             sjj, djj, sjg, djg, sgj, dgj,
             ojj0, ojg0, ogj0, ojj1, ojg1, ogj1,
             accj, accg, idxs, idxd, wv, rows, zv, sem):
        cid = lax.axis_index("c")
        sid = lax.axis_index("s")
        z16 = jnp.zeros((16,), f32)

        def zrow(r, _):
            for cb in range(d // 16):
                zv[r, pl.ds(cb * 16, 16)] = z16
            return 0
        lax.fori_loop(0, ZR, zrow, 0)

        def zero_accs():
            def zj(t, _):
                pltpu.sync_copy(zv, accj.at[pl.ds(sid * RPT_J + t * ZR, ZR)])
                return 0
            lax.fori_loop(0, RPT_J // ZR, zj, 0)
            def zg(t, _):
                pltpu.sync_copy(zv, accg.at[pl.ds(sid * RPT_G + t * ZR, ZR)])
                return 0
            lax.fori_loop(0, RPT_G // ZR, zg, 0)
        zero_accs()
        plsc.subcore_barrier()

        iota16 = lax.iota(jnp.int32, 16)

        def process(feat, w_hbm, s_hbm, d_hbm, acc, nch, weighted):
            base = sid * nch * 8

            def chunk(c2, _):
                r0 = base + c2 * 8
                pltpu.sync_copy(s_hbm.at[pl.ds(r0, 8)], idxs)
                pltpu.sync_copy(d_hbm.at[pl.ds(r0, 8)], idxd)
                if weighted:
                    pltpu.sync_copy(w_hbm.at[pl.ds(r0, 8)], wv)
                for j in range(8):
                    pltpu.async_copy(feat.at[idxs.at[j]], rows, sem).wait()
                    if weighted:
                        def wmul(g, _):
                            e0 = g * 16
                            wv16 = wv[j, pl.ds(e0, 16)]
                            for el in range(16):
                                wb = lax.gather(
                                    wv16,
                                    jnp.full((16, 1), el, jnp.int32),
                                    lax.GatherDimensionNumbers(
                                        offset_dims=(),
                                        collapsed_slice_dims=(0,),
                                        start_index_map=(0,)),
                                    (1,),
                                    mode=lax.GatherScatterMode.PROMISE_IN_BOUNDS)
                                if count_col:
                                    wb = jnp.where(iota16 == d - 1, 1.0, wb)
                                e = e0 + el
                                for cb in range(d // 16):
                                    v = rows[e, pl.ds(cb * 16, 16)]
                                    rows[e, pl.ds(cb * 16, 16)] = v * wb
                            return 0
                        lax.fori_loop(0, 8, wmul, 0)
                    pltpu.sync_copy(rows, acc.at[idxd.at[j]], add=True)
                return 0
            lax.fori_loop(0, nch, chunk, 0)

        def flush(acc, out, rpt):
            pltpu.sync_copy(acc.at[pl.ds(sid * rpt, rpt)],
                            out.at[pl.ds(sid * rpt, rpt)])

        def run_role(fj, fg, wjj, wjg, wgj, ojj, ojg, ogj, weighted):
            process(fj, wjj, sjj, djj, accj, NCH['jj'], weighted)
            process(fj, wjg, sjg, djg, accg, NCH['jg'], weighted)
            plsc.subcore_barrier()
            flush(accj, ojj, RPT_J)
            flush(accg, ojg, RPT_G)
            plsc.subcore_barrier()
            zero_accs()
            plsc.subcore_barrier()
            process(fg, wgj, sgj, dgj, accj, NCH['gj'], weighted)
            plsc.subcore_barrier()
            flush(accj, ogj, RPT_J)

        @pl.when(cid == 0)
        def _():
            run_role(fj0, fg0, w0jj, w0jg, w0gj, ojj0, ojg0, ogj0, weighted0)

        @pl.when(cid == 1)
        def _():
            run_role(fj1, fg1, w1jj, w1jg, w1gj, ojj1, ojg1, ogj1, weighted1)

    return pl.kernel(body, out_type=out_type, mesh=mesh,
                     scratch_types=scratch,
                     compiler_params=pltpu.CompilerParams(
                         use_tc_tiling_on_sc=False))


def _wprep(ep):
    blk = 2048
    grid = ep // blk

    def body(ea_ref, we_e, be_e, we_d, be_d, oe_ref, od_ref):
        ea = ea_ref[...]
        ze = ea[:, 0] * we_e[0, 0] + ea[:, 1] * we_e[0, 1] \
            + ea[:, 2] * we_e[0, 2] + ea[:, 3] * we_e[0, 3] + be_e[0, 0]
        zd = ea[:, 0] * we_d[0, 0] + ea[:, 1] * we_d[0, 1] \
            + ea[:, 2] * we_d[0, 2] + ea[:, 3] * we_d[0, 3] + be_d[0, 0]
        oe_ref[...] = jax.nn.sigmoid(ze).reshape(blk // 128, 128)
        od_ref[...] = jax.nn.sigmoid(zd).reshape(blk // 128, 128)

    return pl.pallas_call(
        body,
        grid=(grid,),
        in_specs=[
            pl.BlockSpec((blk, 4), lambda i: (i, 0)),
            pl.BlockSpec((1, 4), lambda i: (0, 0)),
            pl.BlockSpec((1, 1), lambda i: (0, 0)),
            pl.BlockSpec((1, 4), lambda i: (0, 0)),
            pl.BlockSpec((1, 1), lambda i: (0, 0)),
        ],
        out_specs=[
            pl.BlockSpec((blk // 128, 128), lambda i: (i, 0)),
            pl.BlockSpec((blk // 128, 128), lambda i: (i, 0)),
        ],
        out_shape=[
            jax.ShapeDtypeStruct((ep // 128, 128), jnp.float32),
            jax.ShapeDtypeStruct((ep // 128, 128), jnp.float32),
        ],
    )


def _dense_stage(n_rows, blk, dims, div_flags, n_cnt, has_c, t_out):
    grid = n_rows // blk
    m = len(dims)

    def body(*refs):
        i = 0
        a_refs = refs[i:i + m]; i += m
        w_refs = refs[i:i + m]; i += m
        b_ref = refs[i]; i += 1
        cnt_refs = refs[i:i + n_cnt]; i += n_cnt
        c_ref = None
        if has_c:
            c_ref = refs[i]; i += 1
        if t_out is not None:
            hw_ref = refs[i]; i += 1
            hb_ref = refs[i]; i += 1
        out_refs = refs[i:]

        cnts = [jnp.maximum(cr[...][:, 15:16], 1.0) for cr in cnt_refs]
        z = jnp.broadcast_to(b_ref[...], (blk, 128)).astype(jnp.float32)
        for a_ref, w_ref, df in zip(a_refs, w_refs, div_flags):
            a = a_ref[...]
            if df is not None:
                a = a / cnts[df]
            z = z + jnp.dot(a, w_ref[...],
                            preferred_element_type=jnp.float32)
        zi, zf, zc, zo = (z[:, 0:32], z[:, 32:64], z[:, 64:96], z[:, 96:128])
        cn = jax.nn.sigmoid(zi) * jnp.tanh(zc)
        if has_c:
            cn = cn + jax.nn.sigmoid(zf) * c_ref[...]
        h = jax.nn.sigmoid(zo) * jnp.tanh(cn)
        if t_out is not None:
            out_refs[0][...] = jnp.dot(
                h, hw_ref[...], preferred_element_type=jnp.float32) \
                + hb_ref[...]
        else:
            out_refs[0][...] = h
            out_refs[1][...] = cn

    in_specs = [pl.BlockSpec((blk, di), lambda i: (i, 0)) for di in dims]
    in_specs += [pl.BlockSpec((di, 128), lambda i: (0, 0)) for di in dims]
    in_specs += [pl.BlockSpec((1, 128), lambda i: (0, 0))]
    in_specs += [pl.BlockSpec((blk, 16), lambda i: (i, 0))
                 for _ in range(n_cnt)]
    if has_c:
        in_specs += [pl.BlockSpec((blk, H), lambda i: (i, 0))]
    if t_out is not None:
        in_specs += [pl.BlockSpec((H, t_out), lambda i: (0, 0)),
                     pl.BlockSpec((1, t_out), lambda i: (0, 0))]
        out_specs = [pl.BlockSpec((blk, t_out), lambda i: (i, 0))]
        out_shape = [jax.ShapeDtypeStruct((n_rows, t_out), jnp.float32)]
    else:
        out_specs = [pl.BlockSpec((blk, H), lambda i: (i, 0)),
                     pl.BlockSpec((blk, H), lambda i: (i, 0))]
        out_shape = [jax.ShapeDtypeStruct((n_rows, H), jnp.float32)] * 2

    return pl.pallas_call(body, grid=(grid,), in_specs=in_specs,
                          out_specs=out_specs, out_shape=out_shape)


def _cat_w(p, pre, key, din, pad_to):
    w = jnp.concatenate([p['%s_%s_%s' % (pre, g, key)] for g in GATES], axis=1)
    if pad_to is not None and pad_to != din:
        w = jnp.zeros((pad_to, 4 * H), w.dtype).at[:din].set(w)
    return w


def _bias(p, nt):
    return jnp.concatenate([p['b_%s_%s' % (g, nt)] for g in GATES])[None, :]


def kernel(x_joint, x_grain, ei_jj, ei_jg, ei_gj, ea_jj, ea_jg, ea_gj, params):
    f32 = jnp.float32
    p_e0, p_e1 = params['enc']
    p_d0, p_d1 = params['dec']

    x_pad_j = jnp.zeros((NJ, 16), f32).at[:, :12].set(x_joint).at[:, 15].set(1.0)
    x_pad_g = jnp.zeros((NG, 16), f32).at[:, :10].set(x_grain).at[:, 15].set(1.0)

    ei = {'jj': ei_jj, 'jg': ei_jg, 'gj': ei_gj}
    ea = {'jj': ea_jj, 'jg': ea_jg, 'gj': ea_gj}
    src, dst, eap = {}, {}, {}
    for k in ('jj', 'jg', 'gj'):
        e = ei[k].shape[1]
        pad = EP[k] - e
        src[k] = jnp.pad(ei[k][0], (0, pad)).reshape(EP[k] // 128, 128)
        dst[k] = jnp.pad(ei[k][1], (0, pad),
                         constant_values=NDST[k]).reshape(EP[k] // 128, 128)
        eap[k] = jnp.pad(ea[k], ((0, pad), (0, 0)))

    w_enc, w_dec = {}, {}
    for k in ('jj', 'jg', 'gj'):
        w_enc[k], w_dec[k] = _wprep(EP[k])(
            eap[k], p_e0['we_%s' % k][None, :], p_e0['be_%s' % k][None, None],
            p_d0['we_%s' % k][None, :], p_d0['be_%s' % k][None, None])

    pass1 = _sc_pass(16, True, True, True)
    (aA_jj, aA_jg, aA_gj, aCx_jj, aCx_jg, aCx_gj) = pass1(
        x_pad_j, x_pad_g, x_pad_j, x_pad_g,
        w_enc['jj'], w_enc['jg'], w_enc['gj'],
        w_dec['jj'], w_dec['jg'], w_dec['gj'],
        src['jj'], dst['jj'], src['jg'], dst['jg'], src['gj'], dst['gj'])

    stA_j = _dense_stage(NJ, 2000, [16, 16, 16], [None, 0, 1], 2, False, None)
    stA_g = _dense_stage(NG, 2000, [16, 16], [None, 0], 1, False, None)
    hA_j, cA_j = stA_j(
        x_pad_j, aA_jj, aA_gj,
        _cat_w(p_e0, 'W_x', 'joint', 12, 16),
        _cat_w(p_e0, 'Wm_x', 'jj', 12, 16),
        _cat_w(p_e0, 'Wm_x', 'gj', 10, 16),
        _bias(p_e0, 'joint'), aA_jj, aA_gj)
    hA_g, cA_g = stA_g(
        x_pad_g, aA_jg,
        _cat_w(p_e0, 'W_x', 'grain', 10, 16),
        _cat_w(p_e0, 'Wm_x', 'jg', 12, 16),
        _bias(p_e0, 'grain'), aA_jg)

    pass2 = _sc_pass(32, False, True, False)
    dummy = w_enc
    (aB_jj, aB_jg, aB_gj, aCh_jj, aCh_jg, aCh_gj) = pass2(
        hA_j, hA_g, hA_j, hA_g,
        dummy['jj'], dummy['jg'], dummy['gj'],
        w_dec['jj'], w_dec['jg'], w_dec['gj'],
        src['jj'], dst['jj'], src['jg'], dst['jg'], src['gj'], dst['gj'])

    stB_j = _dense_stage(NJ, 2000, [32, 32, 32], [None, 0, 1], 2, False, None)
    stB_g = _dense_stage(NG, 2000, [32, 32], [None, 0], 1, False, None)
    hB_j, cB_j = stB_j(
        hA_j, aB_jj, aB_gj,
        _cat_w(p_e1, 'W_x', 'joint', 32, None),
        _cat_w(p_e1, 'Wm_x', 'jj', 32, None),
        _cat_w(p_e1, 'Wm_x', 'gj', 32, None),
        _bias(p_e1, 'joint'), aA_jj, aA_gj)
    hB_g, cB_g = stB_g(
        hA_g, aB_jg,
        _cat_w(p_e1, 'W_x', 'grain', 32, None),
        _cat_w(p_e1, 'Wm_x', 'jg', 32, None),
        _bias(p_e1, 'grain'), aA_jg)

    stC_j = _dense_stage(NJ, 2000, [16, 32, 16, 32, 16, 32],
                         [None, None, 0, 0, 1, 1], 2, True, None)
    stC_g = _dense_stage(NG, 2000, [16, 32, 16, 32],
                         [None, None, 0, 0], 1, True, None)
    hC_j, _ = stC_j(
        x_pad_j, hA_j, aCx_jj, aCh_jj, aCx_gj, aCh_gj,
        _cat_w(p_d0, 'W_x', 'joint', 12, 16),
        _cat_w(p_d0, 'W_h', 'joint', 32, None),
        _cat_w(p_d0, 'Wm_x', 'jj', 12, 16),
        _cat_w(p_d0, 'Wm_h', 'jj', 32, None),
        _cat_w(p_d0, 'Wm_x', 'gj', 10, 16),
        _cat_w(p_d0, 'Wm_h', 'gj', 32, None),
        _bias(p_d0, 'joint'), aA_jj, aA_gj, cA_j)
    hC_g, _ = stC_g(
        x_pad_g, hA_g, aCx_jg, aCh_jg,
        _cat_w(p_d0, 'W_x', 'grain', 10, 16),
        _cat_w(p_d0, 'W_h', 'grain', 32, None),
        _cat_w(p_d0, 'Wm_x', 'jg', 12, 16),
        _cat_w(p_d0, 'Wm_h', 'jg', 32, None),
        _bias(p_d0, 'grain'), aA_jg, cA_g)

    pass3 = _sc_pass(32, False, False, False)
    (aD1_jj, aD1_jg, aD1_gj, aD2_jj, aD2_jg, aD2_gj) = pass3(
        hC_j, hC_g, hB_j, hB_g,
        dummy['jj'], dummy['jg'], dummy['gj'],
        dummy['jj'], dummy['jg'], dummy['gj'],
        src['jj'], dst['jj'], src['jg'], dst['jg'], src['gj'], dst['gj'])

    stD_j = _dense_stage(NJ, 2000, [32, 32, 32, 32, 32, 32],
                         [None, None, 0, 0, 1, 1], 2, True, 2)
    stD_g = _dense_stage(NG, 2000, [32, 32, 32, 32],
                         [None, None, 0, 0], 1, True, 4)
    (y_joint,) = stD_j(
        hC_j, hB_j, aD1_jj, aD2_jj, aD1_gj, aD2_gj,
        _cat_w(p_d1, 'W_x', 'joint', 32, None),
        _cat_w(p_d1, 'W_h', 'joint', 32, None),
        _cat_w(p_d1, 'Wm_x', 'jj', 32, None),
        _cat_w(p_d1, 'Wm_h', 'jj', 32, None),
        _cat_w(p_d1, 'Wm_x', 'gj', 32, None),
        _cat_w(p_d1, 'Wm_h', 'gj', 32, None),
        _bias(p_d1, 'joint'), aA_jj, aA_gj, cB_j,
        params['lin']['W_joint'], params['lin']['b_joint'][None, :])
    (y_grain,) = stD_g(
        hC_g, hB_g, aD1_jg, aD2_jg,
        _cat_w(p_d1, 'W_x', 'grain', 32, None),
        _cat_w(p_d1, 'W_h', 'grain', 32, None),
        _cat_w(p_d1, 'Wm_x', 'jg', 32, None),
        _cat_w(p_d1, 'Wm_h', 'jg', 32, None),
        _bias(p_d1, 'grain'), aA_jg, cB_g,
        params['lin']['W_grain'], params['lin']['b_grain'][None, :])
    return (y_joint, y_grain)
